# baseline (speedup 1.0000x reference)
import functools

import jax
import jax.numpy as jnp
from jax import lax
from jax.experimental import pallas as pl
from jax.experimental.pallas import tpu as pltpu
from jax.experimental.pallas import tpu_sc as plsc

NC, NS = 2, 16
NW = NC * NS
S, T = 4, 4096
D = 1024
BPW = (S * T) // NW
WPS = T // BPW

SIZES = [64, 32] * 5 + [32]
OFFS = [0]
for _s in SIZES[:-1]:
    OFFS.append(OFFS[-1] + _s)
BANK = [0 if sz == 64 else 1 for sz in SIZES]
NEXT = {}
for _c, _b in enumerate(BANK):
    for _n in range(_c + 1, len(SIZES)):
        if BANK[_n] == _b:
            NEXT[_c] = _n
            break
    else:
        NEXT[_c] = None

_mesh = plsc.VectorSubcoreMesh(
    core_axis_name="c", subcore_axis_name="s", num_cores=NC, num_subcores=NS
)


@functools.partial(
    pl.kernel,
    out_type=jax.ShapeDtypeStruct((S, T, D), jnp.float32),
    mesh=_mesh,
    scratch_types=[
        pltpu.VMEM((64,), jnp.int32),
        pltpu.VMEM((32,), jnp.int32),
        pltpu.VMEM((64, D), jnp.float32),
        pltpu.VMEM((32, D), jnp.float32),
        pltpu.SemaphoreType.DMA,
        pltpu.SemaphoreType.DMA,
        pltpu.SemaphoreType.DMA,
        pltpu.SemaphoreType.DMA,
        pltpu.SemaphoreType.DMA,
        pltpu.SemaphoreType.DMA,
    ],
)
def _embed(tokens_hbm, table_hbm, out_hbm, idxa_v, idxb_v, rowsa_v, rowsb_v,
           ia, ib, ga, gb, sa, sb):
    idxs = (idxa_v, idxb_v)
    rows = (rowsa_v, rowsb_v)
    isems = (ia, ib)
    gsems = (ga, gb)
    ssems = (sa, sb)
    wid = lax.axis_index("s") * NC + lax.axis_index("c")
    seq = wid // WPS
    col0 = (wid % WPS) * BPW

    def _icopies(c):
        j = BANK[c]
        out = []
        for k in range(SIZES[c] // 32):
            dst = idxs[j] if SIZES[c] == 32 else idxs[j].at[pl.ds(32 * k, 32)]
            out.append(
                pltpu.make_async_copy(
                    tokens_hbm.at[seq, pl.ds(col0 + OFFS[c] + 32 * k, 32)],
                    dst,
                    isems[j],
                )
            )
        return out

    class _Multi:
        def __init__(self, descs):
            self.descs = descs

        def start(self):
            for d in self.descs:
                d.start()

        def wait(self):
            for d in self.descs:
                d.wait()

    def icopy(c):
        return _Multi(_icopies(c))

    def gather(c):
        j = BANK[c]
        return pltpu.make_async_copy(
            table_hbm.at[idxs[j]], rows[j], gsems[j]
        )

    def store(c):
        j = BANK[c]
        return pltpu.make_async_copy(
            rows[j], out_hbm.at[seq, pl.ds(col0 + OFFS[c], SIZES[c])], ssems[j]
        )

    icopy(0).start()
    icopy(1).start()
    for c in range(2):
        icopy(c).wait()
        gather(c).start()

    for c in range(len(SIZES)):
        gather(c).wait()
        nc = NEXT[c]
        if nc is not None:
            icopy(nc).start()
        store(c).start()
        if nc is not None:
            store(c).wait()
            icopy(nc).wait()
            gather(nc).start()

    for c, nc in NEXT.items():
        if nc is None:
            store(c).wait()


def kernel(tokens, W_E):
    return _embed(tokens, W_E)


# speedup vs baseline: 1.0357x; 1.0357x over previous
import functools

import jax
import jax.numpy as jnp
from jax import lax
from jax.experimental import pallas as pl
from jax.experimental.pallas import tpu as pltpu
from jax.experimental.pallas import tpu_sc as plsc

NC, NS = 2, 16
NW = NC * NS
S, T = 4, 4096
D = 1024
BPW = (S * T) // NW
WPS = T // BPW
CHUNK = 56
NFULL = BPW // CHUNK
TAIL = BPW - NFULL * CHUNK
SIZES = [CHUNK] * NFULL + ([TAIL] if TAIL else [])
OFFS = [i * CHUNK for i in range(NFULL)] + ([NFULL * CHUNK] if TAIL else [])

_mesh = plsc.VectorSubcoreMesh(
    core_axis_name="c", subcore_axis_name="s", num_cores=NC, num_subcores=NS
)


@functools.partial(
    pl.kernel,
    out_type=jax.ShapeDtypeStruct((S, T, D), jnp.float32),
    mesh=_mesh,
    scratch_types=[
        pltpu.VMEM((2, CHUNK), jnp.int32),
        pltpu.VMEM((max(TAIL,8),), jnp.int32),
        pltpu.VMEM((2, CHUNK, D), jnp.float32),
        pltpu.VMEM((max(TAIL,8), D), jnp.float32),
        pltpu.SemaphoreType.DMA,
        pltpu.SemaphoreType.DMA,
        pltpu.SemaphoreType.DMA,
        pltpu.SemaphoreType.DMA,
        pltpu.SemaphoreType.DMA,
        pltpu.SemaphoreType.DMA,
        pltpu.SemaphoreType.DMA,
    ],
)
def _embed(tokens_hbm, table_hbm, out_hbm, idx_v, idxt_v, rows_v, rowst_v,
           i0, i1, it, g0, g1, s0, s1):
    isems = (i0, i1)
    gsems = (g0, g1)
    ssems = (s0, s1)
    wid = lax.axis_index("s") * NC + lax.axis_index("c")
    seq = wid // WPS
    col0 = (wid % WPS) * BPW
    nch = len(SIZES)

    def ids_src(c):
        return tokens_hbm.at[pl.ds(seq * T + col0 + OFFS[c], SIZES[c])]

    def idx_ref(c, j):
        return idxt_v if (TAIL and c == nch - 1) else idx_v.at[j]

    def buf(c, j):
        if SIZES[c] == CHUNK:
            return rows_v.at[j]
        return rowst_v.at[pl.ds(0, SIZES[c])]

    def icopy(c, j):
        sem = it if (TAIL and c == nch - 1) else isems[j]
        return pltpu.make_async_copy(ids_src(c), idx_ref(c, j), sem)

    def gather(c, j):
        return pltpu.make_async_copy(
            table_hbm.at[idx_ref(c, j)], buf(c, j), gsems[j]
        )

    def store(c, j):
        return pltpu.make_async_copy(
            buf(c, j), out_hbm.at[seq, pl.ds(col0 + OFFS[c], SIZES[c])], ssems[j]
        )

    icopy(0, 0).start()
    icopy(1, 1).start()
    if TAIL:
        icopy(nch - 1, (nch - 1) % 2).start()
    for j in range(2):
        icopy(j, j).wait()
        gather(j, j).start()

    for c in range(nch):
        j = c % 2
        nc = c + 2
        gather(c, j).wait()
        if nc < nch - 1:
            icopy(nc, j).start()
        store(c, j).start()
        if nc < nch:
            store(c, j).wait()
            icopy(nc, j).wait()
            gather(nc, j).start()

    for c in range(nch - 2, nch):
        store(c, c % 2).wait()


def kernel(tokens, W_E):
    return _embed(tokens.reshape(-1), W_E)
